# baseline (speedup 1.0000x reference)
import functools

import jax
import jax.numpy as jnp
from jax.experimental import pallas as pl
from jax.experimental.pallas import tpu as pltpu

N = 10000
NP = 10240
NG = 128
IT = 512
KB = 2048
AIT = 1024
NI = NP // IT
NK = NP // KB
NAI = NP // AIT
EPS = 1e-5


def _mm_body(x_ref, w_ref, o_ref):
    o_ref[:] = jnp.dot(x_ref[:], w_ref[:], preferred_element_type=jnp.float32)


def _mm(x, w):
    fin, fout = w.shape
    return pl.pallas_call(
        _mm_body,
        grid=(NI,),
        in_specs=[
            pl.BlockSpec((IT, fin), lambda i: (i, 0)),
            pl.BlockSpec((fin, fout), lambda i: (0, 0)),
        ],
        out_specs=pl.BlockSpec((IT, fout), lambda i: (i, 0)),
        out_shape=jax.ShapeDtypeStruct((NP, fout), jnp.float32),
    )(x, w)


def _actmm_body(m_ref, s_ref, t_ref, w_ref, o_ref):
    a = jnp.maximum(m_ref[:] * s_ref[:] + t_ref[:], 0.0)
    o_ref[:] = jnp.dot(a, w_ref[:], preferred_element_type=jnp.float32)


def _actmm(m, s, t, w):
    fin, fout = w.shape
    return pl.pallas_call(
        _actmm_body,
        grid=(NI,),
        in_specs=[
            pl.BlockSpec((IT, fin), lambda i: (i, 0)),
            pl.BlockSpec((1, fin), lambda i: (0, 0)),
            pl.BlockSpec((1, fin), lambda i: (0, 0)),
            pl.BlockSpec((fin, fout), lambda i: (0, 0)),
        ],
        out_specs=pl.BlockSpec((IT, fout), lambda i: (i, 0)),
        out_shape=jax.ShapeDtypeStruct((NP, fout), jnp.float32),
    )(m, s, t, w)


def _agg_body(a_ref, p_ref, o_ref):
    k = pl.program_id(1)

    @pl.when(k == 0)
    def _():
        o_ref[:] = jnp.zeros_like(o_ref)

    a = a_ref[:]
    a_hi = a.astype(jnp.bfloat16)
    a_lo = (a - a_hi.astype(jnp.float32)).astype(jnp.bfloat16)
    p = p_ref[:].astype(jnp.bfloat16)
    o_ref[:] += (jnp.dot(a_hi, p, preferred_element_type=jnp.float32)
                 + jnp.dot(a_lo, p, preferred_element_type=jnp.float32))


def _agg(a, p):
    f = p.shape[1]
    return pl.pallas_call(
        _agg_body,
        grid=(NAI, NK),
        in_specs=[
            pl.BlockSpec((AIT, KB), lambda i, k: (i, k)),
            pl.BlockSpec((KB, f), lambda i, k: (k, 0)),
        ],
        out_specs=pl.BlockSpec((AIT, f), lambda i, k: (i, 0)),
        out_shape=jax.ShapeDtypeStruct((NP, f), jnp.float32),
        compiler_params=pltpu.CompilerParams(
            dimension_semantics=("arbitrary", "arbitrary"),
        ),
    )(a, p)


def _stats_body(m_ref, g_ref, be_ref, s_ref, t_ref, acc_ref):
    i = pl.program_id(0)

    @pl.when(i == 0)
    def _():
        acc_ref[:] = jnp.zeros_like(acc_ref)

    m = m_ref[:]
    acc_ref[0:1] += jnp.sum(m, axis=0, keepdims=True)
    acc_ref[1:2] += jnp.sum(m * m, axis=0, keepdims=True)

    @pl.when(i == NI - 1)
    def _():
        mean = acc_ref[0:1] / float(N)
        var = acc_ref[1:2] / float(N) - mean * mean
        s = g_ref[:] * jax.lax.rsqrt(var + EPS)
        s_ref[:] = s
        t_ref[:] = be_ref[:] - mean * s


def _stats(m, g, be):
    f = m.shape[1]
    return pl.pallas_call(
        _stats_body,
        grid=(NI,),
        in_specs=[
            pl.BlockSpec((IT, f), lambda i: (i, 0)),
            pl.BlockSpec((1, f), lambda i: (0, 0)),
            pl.BlockSpec((1, f), lambda i: (0, 0)),
        ],
        out_specs=[
            pl.BlockSpec((1, f), lambda i: (0, 0)),
            pl.BlockSpec((1, f), lambda i: (0, 0)),
        ],
        out_shape=[
            jax.ShapeDtypeStruct((1, f), jnp.float32),
            jax.ShapeDtypeStruct((1, f), jnp.float32),
        ],
        scratch_shapes=[pltpu.VMEM((2, f), jnp.float32)],
    )(m, g.reshape(1, f), be.reshape(1, f))


def _pool_body(b_ref, m_ref, s_ref, t_ref, x_ref, pooled_ref, xmax_ref):
    tile = pl.program_id(0)
    fout = m_ref.shape[1]
    fin = x_ref.shape[1]
    for j in range(8):
        g = tile * 8 + j
        st = b_ref[g]
        en = b_ref[g + 1]
        c0 = st // 8
        c1 = (en + 7) // 8

        def body(c, carry):
            acch, accx = carry
            base = c * 8
            rid = base + jax.lax.broadcasted_iota(jnp.int32, (8, 1), 0)
            mask = (rid >= st) & (rid < en)
            h = jnp.maximum(m_ref[pl.ds(base, 8), :] * s_ref[:] + t_ref[:], 0.0)
            acch = jnp.maximum(acch, jnp.where(mask, h, -jnp.inf))
            accx = jnp.maximum(accx,
                               jnp.where(mask, x_ref[pl.ds(base, 8), :], -jnp.inf))
            return acch, accx

        acch, accx = jax.lax.fori_loop(
            c0, c1, body,
            (jnp.full((8, fout), -jnp.inf, jnp.float32),
             jnp.full((8, fin), -jnp.inf, jnp.float32)))
        pooled_ref[pl.ds(j, 1), :] = jnp.max(acch, axis=0, keepdims=True)
        xmax_ref[pl.ds(j, 1), :] = jnp.max(accx, axis=0, keepdims=True)


def _pool(bounds, m4, s, t, x):
    f = m4.shape[1]
    fin = x.shape[1]
    return pl.pallas_call(
        _pool_body,
        grid=(NG // 8,),
        in_specs=[
            pl.BlockSpec(memory_space=pltpu.SMEM),
            pl.BlockSpec((NP, f), lambda i: (0, 0)),
            pl.BlockSpec((1, f), lambda i: (0, 0)),
            pl.BlockSpec((1, f), lambda i: (0, 0)),
            pl.BlockSpec((NP, fin), lambda i: (0, 0)),
        ],
        out_specs=[
            pl.BlockSpec((8, f), lambda i: (i, 0)),
            pl.BlockSpec((8, fin), lambda i: (i, 0)),
        ],
        out_shape=[
            jax.ShapeDtypeStruct((NG, f), jnp.float32),
            jax.ShapeDtypeStruct((NG, fin), jnp.float32),
        ],
    )(bounds, m4, s, t, x)


def _final_body(p_ref, xm_ref, wr_ref, br_ref, o_ref):
    o_ref[:] = (p_ref[:]
                + jnp.dot(xm_ref[:], wr_ref[:], preferred_element_type=jnp.float32)
                + br_ref[:])


def _final(pooled, xmax, wr, br):
    fin, fout = wr.shape
    return pl.pallas_call(
        _final_body,
        in_specs=[
            pl.BlockSpec((NG, fout), lambda: (0, 0)),
            pl.BlockSpec((NG, fin), lambda: (0, 0)),
            pl.BlockSpec((fin, fout), lambda: (0, 0)),
            pl.BlockSpec((1, fout), lambda: (0, 0)),
        ],
        out_specs=pl.BlockSpec((NG, fout), lambda: (0, 0)),
        out_shape=jax.ShapeDtypeStruct((NG, fout), jnp.float32),
    )(pooled, xmax, wr, br.reshape(1, fout))


def kernel(x, edge_index, batch, W1, b1, g1, be1, W2, b2, g2, be2,
           W3, b3, g3, be3, W4, b4, g4, be4, Wr, br):
    ei = edge_index.astype(jnp.int32)
    loop = jnp.arange(N, dtype=jnp.int32)
    src = jnp.concatenate([ei[0], loop])
    dst = jnp.concatenate([ei[1], loop])
    deg = jnp.zeros((NP,), jnp.float32).at[dst].add(1.0)
    dis = jnp.where(deg > 0, jax.lax.rsqrt(jnp.maximum(deg, 1.0)), 0.0)
    normv = dis[src] * dis[dst]
    a_hat = jnp.zeros((NP, NP), jnp.float32).at[dst, src].add(normv)

    xp = jnp.pad(x.astype(jnp.float32), ((0, NP - N), (0, 0)))
    bounds = jnp.searchsorted(batch.astype(jnp.int32),
                              jnp.arange(NG + 1, dtype=jnp.int32)).astype(jnp.int32)

    p = _mm(xp, W1)
    m = _agg(a_hat, p)
    s, t = _stats(m, g1, be1)

    p = _actmm(m, s, t, W2)
    m = _agg(a_hat, p)
    s, t = _stats(m, g2, be2)

    p = _actmm(m, s, t, W3)
    m = _agg(a_hat, p)
    s, t = _stats(m, g3, be3)

    p = _actmm(m, s, t, W4)
    m = _agg(a_hat, p)
    s, t = _stats(m, g4, be4)

    pooled, xmax = _pool(bounds, m, s, t, xp)
    return _final(pooled, xmax, Wr, br)


# speedup vs baseline: 1.0462x; 1.0462x over previous
import functools

import jax
import jax.numpy as jnp
from jax.experimental import pallas as pl
from jax.experimental.pallas import tpu as pltpu

N = 10000
NP = 10240
NG = 128
IT = 512
KB = 2048
AIT = 1024
NI = NP // IT
NK = NP // KB
NAI = NP // AIT
EPS = 1e-5


def _mm_body(x_ref, w_ref, o_ref):
    o_ref[:] = jnp.dot(x_ref[:], w_ref[:], preferred_element_type=jnp.float32)


def _mm(x, w):
    fin, fout = w.shape
    return pl.pallas_call(
        _mm_body,
        grid=(NI,),
        in_specs=[
            pl.BlockSpec((IT, fin), lambda i: (i, 0)),
            pl.BlockSpec((fin, fout), lambda i: (0, 0)),
        ],
        out_specs=pl.BlockSpec((IT, fout), lambda i: (i, 0)),
        out_shape=jax.ShapeDtypeStruct((NP, fout), jnp.float32),
    )(x, w)


def _actmm_body(m_ref, s_ref, t_ref, w_ref, o_ref):
    a = jnp.maximum(m_ref[:] * s_ref[:] + t_ref[:], 0.0)
    o_ref[:] = jnp.dot(a, w_ref[:], preferred_element_type=jnp.float32)


def _actmm(m, s, t, w):
    fin, fout = w.shape
    return pl.pallas_call(
        _actmm_body,
        grid=(NI,),
        in_specs=[
            pl.BlockSpec((IT, fin), lambda i: (i, 0)),
            pl.BlockSpec((1, fin), lambda i: (0, 0)),
            pl.BlockSpec((1, fin), lambda i: (0, 0)),
            pl.BlockSpec((fin, fout), lambda i: (0, 0)),
        ],
        out_specs=pl.BlockSpec((IT, fout), lambda i: (i, 0)),
        out_shape=jax.ShapeDtypeStruct((NP, fout), jnp.float32),
    )(m, s, t, w)


def _agg_body(a_ref, p_ref, o_ref):
    k = pl.program_id(1)

    @pl.when(k == 0)
    def _():
        o_ref[:] = jnp.zeros_like(o_ref)

    o_ref[:] += jnp.dot(a_ref[:], p_ref[:], preferred_element_type=jnp.float32)


def _agg(a, p):
    f = p.shape[1]
    return pl.pallas_call(
        _agg_body,
        grid=(NAI, NK),
        in_specs=[
            pl.BlockSpec((AIT, KB), lambda i, k: (i, k)),
            pl.BlockSpec((KB, f), lambda i, k: (k, 0)),
        ],
        out_specs=pl.BlockSpec((AIT, f), lambda i, k: (i, 0)),
        out_shape=jax.ShapeDtypeStruct((NP, f), jnp.float32),
        compiler_params=pltpu.CompilerParams(
            dimension_semantics=("arbitrary", "arbitrary"),
        ),
    )(a, p)


def _stats_body(m_ref, g_ref, be_ref, s_ref, t_ref, acc_ref):
    i = pl.program_id(0)

    @pl.when(i == 0)
    def _():
        acc_ref[:] = jnp.zeros_like(acc_ref)

    m = m_ref[:]
    acc_ref[0:1] += jnp.sum(m, axis=0, keepdims=True)
    acc_ref[1:2] += jnp.sum(m * m, axis=0, keepdims=True)

    @pl.when(i == NI - 1)
    def _():
        mean = acc_ref[0:1] / float(N)
        var = acc_ref[1:2] / float(N) - mean * mean
        s = g_ref[:] * jax.lax.rsqrt(var + EPS)
        s_ref[:] = s
        t_ref[:] = be_ref[:] - mean * s


def _stats(m, g, be):
    f = m.shape[1]
    return pl.pallas_call(
        _stats_body,
        grid=(NI,),
        in_specs=[
            pl.BlockSpec((IT, f), lambda i: (i, 0)),
            pl.BlockSpec((1, f), lambda i: (0, 0)),
            pl.BlockSpec((1, f), lambda i: (0, 0)),
        ],
        out_specs=[
            pl.BlockSpec((1, f), lambda i: (0, 0)),
            pl.BlockSpec((1, f), lambda i: (0, 0)),
        ],
        out_shape=[
            jax.ShapeDtypeStruct((1, f), jnp.float32),
            jax.ShapeDtypeStruct((1, f), jnp.float32),
        ],
        scratch_shapes=[pltpu.VMEM((2, f), jnp.float32)],
    )(m, g.reshape(1, f), be.reshape(1, f))


def _pool_body(b_ref, m_ref, s_ref, t_ref, x_ref, pooled_ref, xmax_ref):
    tile = pl.program_id(0)
    fout = m_ref.shape[1]
    fin = x_ref.shape[1]
    for j in range(8):
        g = tile * 8 + j
        st = b_ref[g]
        en = b_ref[g + 1]
        c0 = st // 8
        c1 = (en + 7) // 8

        def body(c, carry):
            acch, accx = carry
            base = c * 8
            rid = base + jax.lax.broadcasted_iota(jnp.int32, (8, 1), 0)
            mask = (rid >= st) & (rid < en)
            h = jnp.maximum(m_ref[pl.ds(base, 8), :] * s_ref[:] + t_ref[:], 0.0)
            acch = jnp.maximum(acch, jnp.where(mask, h, -jnp.inf))
            accx = jnp.maximum(accx,
                               jnp.where(mask, x_ref[pl.ds(base, 8), :], -jnp.inf))
            return acch, accx

        acch, accx = jax.lax.fori_loop(
            c0, c1, body,
            (jnp.full((8, fout), -jnp.inf, jnp.float32),
             jnp.full((8, fin), -jnp.inf, jnp.float32)))
        pooled_ref[pl.ds(j, 1), :] = jnp.max(acch, axis=0, keepdims=True)
        xmax_ref[pl.ds(j, 1), :] = jnp.max(accx, axis=0, keepdims=True)


def _pool(bounds, m4, s, t, x):
    f = m4.shape[1]
    fin = x.shape[1]
    return pl.pallas_call(
        _pool_body,
        grid=(NG // 8,),
        in_specs=[
            pl.BlockSpec(memory_space=pltpu.SMEM),
            pl.BlockSpec((NP, f), lambda i: (0, 0)),
            pl.BlockSpec((1, f), lambda i: (0, 0)),
            pl.BlockSpec((1, f), lambda i: (0, 0)),
            pl.BlockSpec((NP, fin), lambda i: (0, 0)),
        ],
        out_specs=[
            pl.BlockSpec((8, f), lambda i: (i, 0)),
            pl.BlockSpec((8, fin), lambda i: (i, 0)),
        ],
        out_shape=[
            jax.ShapeDtypeStruct((NG, f), jnp.float32),
            jax.ShapeDtypeStruct((NG, fin), jnp.float32),
        ],
    )(bounds, m4, s, t, x)


def _final_body(p_ref, xm_ref, wr_ref, br_ref, o_ref):
    o_ref[:] = (p_ref[:]
                + jnp.dot(xm_ref[:], wr_ref[:], preferred_element_type=jnp.float32)
                + br_ref[:])


def _final(pooled, xmax, wr, br):
    fin, fout = wr.shape
    return pl.pallas_call(
        _final_body,
        in_specs=[
            pl.BlockSpec((NG, fout), lambda: (0, 0)),
            pl.BlockSpec((NG, fin), lambda: (0, 0)),
            pl.BlockSpec((fin, fout), lambda: (0, 0)),
            pl.BlockSpec((1, fout), lambda: (0, 0)),
        ],
        out_specs=pl.BlockSpec((NG, fout), lambda: (0, 0)),
        out_shape=jax.ShapeDtypeStruct((NG, fout), jnp.float32),
    )(pooled, xmax, wr, br.reshape(1, fout))


def kernel(x, edge_index, batch, W1, b1, g1, be1, W2, b2, g2, be2,
           W3, b3, g3, be3, W4, b4, g4, be4, Wr, br):
    ei = edge_index.astype(jnp.int32)
    loop = jnp.arange(N, dtype=jnp.int32)
    src = jnp.concatenate([ei[0], loop])
    dst = jnp.concatenate([ei[1], loop])
    deg = jnp.zeros((NP,), jnp.float32).at[dst].add(1.0)
    dis = jnp.where(deg > 0, jax.lax.rsqrt(jnp.maximum(deg, 1.0)), 0.0)
    normv = dis[src] * dis[dst]
    a_hat = jnp.zeros((NP, NP), jnp.float32).at[dst, src].add(normv)

    xp = jnp.pad(x.astype(jnp.float32), ((0, NP - N), (0, 0)))
    bounds = jnp.searchsorted(batch.astype(jnp.int32),
                              jnp.arange(NG + 1, dtype=jnp.int32)).astype(jnp.int32)

    p = _mm(xp, W1)
    m = _agg(a_hat, p)
    s, t = _stats(m, g1, be1)

    p = _actmm(m, s, t, W2)
    m = _agg(a_hat, p)
    s, t = _stats(m, g2, be2)

    p = _actmm(m, s, t, W3)
    m = _agg(a_hat, p)
    s, t = _stats(m, g3, be3)

    p = _actmm(m, s, t, W4)
    m = _agg(a_hat, p)
    s, t = _stats(m, g4, be4)

    pooled, xmax = _pool(bounds, m, s, t, xp)
    return _final(pooled, xmax, Wr, br)
